# baseline (speedup 1.0000x reference)
import functools

import jax
import jax.numpy as jnp
from jax import lax
from jax.experimental import pallas as pl
from jax.experimental.pallas import tpu as pltpu
from jax.experimental.pallas import tpu_sc as plsc

N = 10000
E = 320000
D_IN = 128
H = 64
C = 16
NG = 64

NC = 2
NS = 16
NW = NC * NS
CHUNK = 128
EPC0 = 80
EPC1 = 80
EPCM = max(EPC0, EPC1)
E_PAD = NS * (EPC0 + EPC1) * CHUNK
NP = 10240
SLAB = NP // NS

_mesh = plsc.VectorSubcoreMesh(core_axis_name="c", subcore_axis_name="s")


@functools.partial(
    pl.kernel,
    out_type=jax.ShapeDtypeStruct((NC, NP), jnp.float32),
    mesh=_mesh,
    scratch_types=[
        pltpu.VMEM((EPCM, CHUNK), jnp.int32),
        pltpu.VMEM((CHUNK,), jnp.float32),
        pltpu.VMEM((SLAB,), jnp.float32),
        pltpu.VMEM_SHARED((NP,), jnp.float32),
    ],
)
def _deg_call(dsts_hbm, out_hbm, idx_d, ones_v, buf, acc):
    cid = lax.axis_index("c")
    sid = lax.axis_index("s")
    wid = cid * NS + sid

    def zrow(i, carry):
        buf[pl.ds(i * 16, 16)] = jnp.zeros((16,), jnp.float32)
        return carry

    lax.fori_loop(0, SLAB // 16, zrow, 0)
    for i in range(CHUNK // 16):
        ones_v[pl.ds(i * 16, 16)] = jnp.ones((16,), jnp.float32)
    pltpu.sync_copy(buf, acc.at[pl.ds(sid * SLAB, SLAB)])
    plsc.subcore_barrier()

    pltpu.sync_copy(dsts_hbm.at[wid], idx_d)

    def step(j, carry):
        pltpu.sync_copy(ones_v, acc.at[idx_d.at[j]], add=True)
        return carry

    lax.fori_loop(0, EPC0, step, 0)
    plsc.subcore_barrier()
    pltpu.sync_copy(acc.at[pl.ds(sid * SLAB, SLAB)], buf)
    pltpu.sync_copy(buf, out_hbm.at[cid, pl.ds(sid * SLAB, SLAB)])


@functools.partial(
    pl.kernel,
    out_type=jax.ShapeDtypeStruct((NC, NP, H), jnp.float32),
    mesh=_mesh,
    compiler_params=pltpu.CompilerParams(use_tc_tiling_on_sc=False),
    scratch_types=[
        pltpu.VMEM((EPCM, CHUNK), jnp.int32),
        pltpu.VMEM((EPCM, CHUNK), jnp.int32),
        pltpu.VMEM((CHUNK, H), jnp.float32),
        pltpu.VMEM((SLAB, H), jnp.float32),
        pltpu.VMEM_SHARED((NP, H), jnp.float32),
        pltpu.SemaphoreType.DMA,
    ],
)
def _spmm_call(table_hbm, srcs_hbm, dsts_hbm, out_hbm,
               idx_s, idx_d, rows, buf, acc, sem):
    cid = lax.axis_index("c")
    sid = lax.axis_index("s")
    wid = cid * NS + sid

    def zrow(r, carry):
        for c4 in range(H // 16):
            buf[r, pl.ds(c4 * 16, 16)] = jnp.zeros((16,), jnp.float32)
        return carry

    lax.fori_loop(0, SLAB, zrow, 0)
    pltpu.sync_copy(buf, acc.at[pl.ds(sid * SLAB, SLAB)])
    plsc.subcore_barrier()

    pltpu.sync_copy(srcs_hbm.at[wid], idx_s)
    pltpu.sync_copy(dsts_hbm.at[wid], idx_d)

    def step(j, carry):
        pltpu.async_copy(table_hbm.at[idx_s.at[j]], rows, sem).wait()
        pltpu.sync_copy(rows, acc.at[idx_d.at[j]], add=True)
        return carry

    lax.fori_loop(0, EPC0, step, 0)
    plsc.subcore_barrier()
    pltpu.sync_copy(acc.at[pl.ds(sid * SLAB, SLAB)], buf)
    pltpu.sync_copy(buf, out_hbm.at[cid, pl.ds(sid * SLAB, SLAB)])


def _prep_body(x_ref, w_ref, degc_ref, y_ref, dinv_ref):
    deg = degc_ref[0] + degc_ref[1] + 1.0
    dinv = lax.rsqrt(deg)
    xw = jnp.dot(x_ref[...], w_ref[...], preferred_element_type=jnp.float32)
    y_ref[0:N, :] = xw * dinv[0:N]
    y_ref[N:NP, :] = jnp.zeros((NP - N, H), jnp.float32)
    dinv_ref[...] = dinv


def _mid_body(z_ref, y1_ref, dinv_ref, b1_ref, w2_ref, y2_ref):
    dinv = dinv_ref[...]
    h1 = jnp.maximum(
        (z_ref[0] + z_ref[1] + y1_ref[...]) * dinv + b1_ref[...], 0.0)
    y2_ref[...] = jnp.dot(h1, w2_ref[...],
                          preferred_element_type=jnp.float32) * dinv


def _fin_body(z_ref, y2_ref, dinv_ref, b2_ref, batch_ref, wl_ref, bl_ref,
              o_ref):
    h2 = jnp.maximum(
        (z_ref[0] + z_ref[1] + y2_ref[...]) * dinv_ref[...] + b2_ref[...],
        0.0)
    gids = lax.broadcasted_iota(jnp.int32, (NG, NP), 0)
    mask = jnp.where(gids == batch_ref[...], 1.0, 0.0)
    pooled = jnp.dot(mask, h2, preferred_element_type=jnp.float32)
    logits = jnp.dot(pooled, wl_ref[...],
                     preferred_element_type=jnp.float32) + bl_ref[...]
    m = jnp.max(logits, axis=1, keepdims=True)
    s = logits - m
    o_ref[...] = s - jnp.log(jnp.sum(jnp.exp(s), axis=1, keepdims=True))


def _tc(body, out_shape):
    return pl.pallas_call(body, out_shape=out_shape)


def _slabify(v):
    flat = jnp.concatenate([v, jnp.full((E_PAD - E,), N, jnp.int32)])
    n0 = NS * EPC0 * CHUNK
    c0 = flat[:n0].reshape(NS, EPC0, CHUNK)
    c1 = flat[n0:].reshape(NS, EPC1, CHUNK)
    c0 = jnp.pad(c0, ((0, 0), (0, EPCM - EPC0), (0, 0)), constant_values=N)
    c1 = jnp.pad(c1, ((0, 0), (0, EPCM - EPC1), (0, 0)), constant_values=N)
    return jnp.concatenate([c0, c1], axis=0)


def kernel(x, edge_index, batch, W1, b1, W2, b2, Wl, bl):
    f32 = jnp.float32
    srcs = _slabify(edge_index[0])
    dsts = _slabify(edge_index[1])
    batch_p = jnp.concatenate(
        [batch, jnp.full((NP - N,), NG, jnp.int32)]).reshape(1, NP)
    b1r = b1.reshape(1, H)
    b2r = b2.reshape(1, H)
    blr = bl.reshape(1, C)

    degp = _deg_call(dsts)
    degc = degp.reshape(NC, NP, 1)
    y1, dinv = _tc(_prep_body,
                   (jax.ShapeDtypeStruct((NP, H), f32),
                    jax.ShapeDtypeStruct((NP, 1), f32)))(x, W1, degc)
    z1 = _spmm_call(y1, srcs, dsts)
    y2 = _tc(_mid_body, jax.ShapeDtypeStruct((NP, H), f32))(
        z1, y1, dinv, b1r, W2)
    z2 = _spmm_call(y2, srcs, dsts)
    out = _tc(_fin_body, jax.ShapeDtypeStruct((NG, C), f32))(
        z2, y2, dinv, b2r, batch_p, Wl, blr)
    return out


# speedup vs baseline: 1.5095x; 1.5095x over previous
import functools

import jax
import jax.numpy as jnp
from jax import lax
from jax.experimental import pallas as pl
from jax.experimental.pallas import tpu as pltpu
from jax.experimental.pallas import tpu_sc as plsc

N = 10000
E = 320000
D_IN = 128
H = 64
C = 16
NG = 64

NC = 2
NS = 16
NW = NC * NS
CHUNK = 128
EPC = 79
EPW = EPC * CHUNK
E_PAD = NW * EPW
NP = 10240
SLAB = NP // NS

_mesh = plsc.VectorSubcoreMesh(core_axis_name="c", subcore_axis_name="s")


@functools.partial(
    pl.kernel,
    out_type=jax.ShapeDtypeStruct((NC, NP), jnp.float32),
    mesh=_mesh,
    scratch_types=[
        pltpu.VMEM((EPC, CHUNK), jnp.int32),
        pltpu.VMEM((CHUNK,), jnp.float32),
        pltpu.VMEM((SLAB,), jnp.float32),
        pltpu.VMEM_SHARED((NP,), jnp.float32),
    ],
)
def _deg_call(dsts_hbm, out_hbm, idx_d, ones_v, buf, acc):
    cid = lax.axis_index("c")
    sid = lax.axis_index("s")
    wid = cid * NS + sid

    def zrow(i, carry):
        buf[pl.ds(i * 16, 16)] = jnp.zeros((16,), jnp.float32)
        return carry

    lax.fori_loop(0, SLAB // 16, zrow, 0)
    for i in range(CHUNK // 16):
        ones_v[pl.ds(i * 16, 16)] = jnp.ones((16,), jnp.float32)
    pltpu.sync_copy(buf, acc.at[pl.ds(sid * SLAB, SLAB)])
    plsc.subcore_barrier()

    pltpu.sync_copy(dsts_hbm.at[wid], idx_d)

    def step(j, carry):
        pltpu.sync_copy(ones_v, acc.at[idx_d.at[j]], add=True)
        return carry

    lax.fori_loop(0, EPC, step, 0)
    plsc.subcore_barrier()
    pltpu.sync_copy(acc.at[pl.ds(sid * SLAB, SLAB)], buf)
    pltpu.sync_copy(buf, out_hbm.at[cid, pl.ds(sid * SLAB, SLAB)])


@functools.partial(
    pl.kernel,
    out_type=jax.ShapeDtypeStruct((NC, NP, H), jnp.float32),
    mesh=_mesh,
    compiler_params=pltpu.CompilerParams(use_tc_tiling_on_sc=False),
    scratch_types=[
        pltpu.VMEM((EPC, CHUNK), jnp.int32),
        pltpu.VMEM((EPC, CHUNK), jnp.int32),
        pltpu.VMEM((CHUNK, H), jnp.float32),
        pltpu.VMEM((SLAB, H), jnp.float32),
        pltpu.VMEM_SHARED((NP, H), jnp.float32),
        pltpu.SemaphoreType.DMA,
    ],
)
def _spmm_call(table_hbm, srcs_hbm, dsts_hbm, out_hbm,
               idx_s, idx_d, rows, buf, acc, sem):
    cid = lax.axis_index("c")
    sid = lax.axis_index("s")
    wid = cid * NS + sid

    def zrow(r, carry):
        for c4 in range(H // 16):
            buf[r, pl.ds(c4 * 16, 16)] = jnp.zeros((16,), jnp.float32)
        return carry

    lax.fori_loop(0, SLAB, zrow, 0)
    pltpu.sync_copy(buf, acc.at[pl.ds(sid * SLAB, SLAB)])
    plsc.subcore_barrier()

    pltpu.sync_copy(srcs_hbm.at[wid], idx_s)
    pltpu.sync_copy(dsts_hbm.at[wid], idx_d)

    def step(j, carry):
        pltpu.async_copy(table_hbm.at[idx_s.at[j]], rows, sem).wait()
        pltpu.sync_copy(rows, acc.at[idx_d.at[j]], add=True)
        return carry

    lax.fori_loop(0, EPC, step, 0)
    plsc.subcore_barrier()
    pltpu.sync_copy(acc.at[pl.ds(sid * SLAB, SLAB)], buf)
    pltpu.sync_copy(buf, out_hbm.at[cid, pl.ds(sid * SLAB, SLAB)])


def _mm_body(x_ref, w_ref, o_ref):
    o_ref[...] = jnp.dot(x_ref[...], w_ref[...],
                         preferred_element_type=jnp.float32)


def _prep_body(xw_ref, degc_ref, y_ref, dinv_ref):
    deg = degc_ref[0] + degc_ref[1] + 1.0
    dinv = lax.rsqrt(deg)
    y_ref[...] = xw_ref[...] * dinv
    dinv_ref[...] = dinv


def _mid_body(z_ref, y1_ref, dinv_ref, b1_ref, w2_ref, y2_ref):
    dinv = dinv_ref[...]
    h1 = jnp.maximum(
        (z_ref[0] + z_ref[1] + y1_ref[...]) * dinv + b1_ref[...], 0.0)
    y2_ref[...] = jnp.dot(h1, w2_ref[...],
                          preferred_element_type=jnp.float32) * dinv


def _fin_body(z_ref, y2_ref, dinv_ref, b2_ref, batch_ref, wl_ref, bl_ref,
              o_ref):
    h2 = jnp.maximum(
        (z_ref[0] + z_ref[1] + y2_ref[...]) * dinv_ref[...] + b2_ref[...],
        0.0)
    gids = lax.broadcasted_iota(jnp.int32, (NG, NP), 0)
    mask = jnp.where(gids == batch_ref[...], 1.0, 0.0)
    pooled = jnp.dot(mask, h2, preferred_element_type=jnp.float32)
    logits = jnp.dot(pooled, wl_ref[...],
                     preferred_element_type=jnp.float32) + bl_ref[...]
    m = jnp.max(logits, axis=1, keepdims=True)
    s = logits - m
    o_ref[...] = s - jnp.log(jnp.sum(jnp.exp(s), axis=1, keepdims=True))


def _tc(body, out_shape):
    return pl.pallas_call(body, out_shape=out_shape)


def kernel(x, edge_index, batch, W1, b1, W2, b2, Wl, bl):
    f32 = jnp.float32
    src = edge_index[0]
    dst = edge_index[1]
    pad = jnp.full((E_PAD - E,), N, jnp.int32)
    srcs = jnp.concatenate([src, pad]).reshape(NW, EPC, CHUNK)
    dsts = jnp.concatenate([dst, pad]).reshape(NW, EPC, CHUNK)
    x_p = jnp.pad(x, ((0, NP - N), (0, 0)))
    batch_p = jnp.concatenate(
        [batch, jnp.full((NP - N,), NG, jnp.int32)]).reshape(1, NP)
    b1r = b1.reshape(1, H)
    b2r = b2.reshape(1, H)
    blr = bl.reshape(1, C)

    degp = _deg_call(dsts)
    degc = degp.reshape(NC, NP, 1)
    xw1 = _tc(_mm_body, jax.ShapeDtypeStruct((NP, H), f32))(x_p, W1)
    y1, dinv = _tc(_prep_body,
                   (jax.ShapeDtypeStruct((NP, H), f32),
                    jax.ShapeDtypeStruct((NP, 1), f32)))(xw1, degc)
    z1 = _spmm_call(y1, srcs, dsts)
    y2 = _tc(_mid_body, jax.ShapeDtypeStruct((NP, H), f32))(
        z1, y1, dinv, b1r, W2)
    z2 = _spmm_call(y2, srcs, dsts)
    out = _tc(_fin_body, jax.ShapeDtypeStruct((NG, C), f32))(
        z2, y2, dinv, b2r, batch_p, Wl, blr)
    return out


# speedup vs baseline: 1.5125x; 1.0020x over previous
import functools

import jax
import jax.numpy as jnp
from jax import lax
from jax.experimental import pallas as pl
from jax.experimental.pallas import tpu as pltpu
from jax.experimental.pallas import tpu_sc as plsc

N = 10000
E = 320000
D_IN = 128
H = 64
C = 16
NG = 64

NC = 2
NS = 16
NW = NC * NS
CHUNK = 128
EPC = 79
EPW = EPC * CHUNK
E_PAD = NW * EPW
NP = 10240
SLAB = NP // NS

_mesh = plsc.VectorSubcoreMesh(core_axis_name="c", subcore_axis_name="s")


@functools.partial(
    pl.kernel,
    out_type=jax.ShapeDtypeStruct((NC, NP), jnp.float32),
    mesh=_mesh,
    scratch_types=[
        pltpu.VMEM((EPC, CHUNK), jnp.int32),
        pltpu.VMEM((CHUNK,), jnp.float32),
        pltpu.VMEM((SLAB,), jnp.float32),
        pltpu.VMEM_SHARED((NP,), jnp.float32),
    ],
)
def _deg_call(dsts_hbm, out_hbm, idx_d, ones_v, buf, acc):
    cid = lax.axis_index("c")
    sid = lax.axis_index("s")
    wid = cid * NS + sid

    def zrow(i, carry):
        buf[pl.ds(i * 16, 16)] = jnp.zeros((16,), jnp.float32)
        return carry

    lax.fori_loop(0, SLAB // 16, zrow, 0)
    for i in range(CHUNK // 16):
        ones_v[pl.ds(i * 16, 16)] = jnp.ones((16,), jnp.float32)
    pltpu.sync_copy(buf, acc.at[pl.ds(sid * SLAB, SLAB)])
    plsc.subcore_barrier()

    pltpu.sync_copy(dsts_hbm.at[wid], idx_d)

    def step(j, carry):
        pltpu.sync_copy(ones_v, acc.at[idx_d.at[j]], add=True)
        return carry

    lax.fori_loop(0, EPC, step, 0)
    plsc.subcore_barrier()
    pltpu.sync_copy(acc.at[pl.ds(sid * SLAB, SLAB)], buf)
    pltpu.sync_copy(buf, out_hbm.at[cid, pl.ds(sid * SLAB, SLAB)])


@functools.partial(
    pl.kernel,
    out_type=jax.ShapeDtypeStruct((NC, NP, H), jnp.float32),
    mesh=_mesh,
    compiler_params=pltpu.CompilerParams(use_tc_tiling_on_sc=False),
    scratch_types=[
        pltpu.VMEM((EPC, CHUNK), jnp.int32),
        pltpu.VMEM((EPC, CHUNK), jnp.int32),
        pltpu.VMEM((CHUNK, H), jnp.float32),
        pltpu.VMEM((SLAB, H), jnp.float32),
        pltpu.VMEM_SHARED((NP, H), jnp.float32),
        pltpu.SemaphoreType.DMA,
    ],
)
def _spmm_call(table_hbm, srcs_hbm, dsts_hbm, out_hbm,
               idx_s, idx_d, rows, buf, acc, sem):
    cid = lax.axis_index("c")
    sid = lax.axis_index("s")
    wid = cid * NS + sid

    def zrow(r, carry):
        for c4 in range(H // 16):
            buf[r, pl.ds(c4 * 16, 16)] = jnp.zeros((16,), jnp.float32)
        return carry

    lax.fori_loop(0, SLAB, zrow, 0)
    pltpu.sync_copy(buf, acc.at[pl.ds(sid * SLAB, SLAB)])
    plsc.subcore_barrier()

    pltpu.sync_copy(srcs_hbm.at[wid], idx_s)
    pltpu.sync_copy(dsts_hbm.at[wid], idx_d)

    def step(j, carry):
        pltpu.async_copy(table_hbm.at[idx_s.at[j]], rows, sem).wait()
        pltpu.sync_copy(rows, acc.at[idx_d.at[j]], add=True)
        return carry

    lax.fori_loop(0, EPC, step, 0)
    plsc.subcore_barrier()
    pltpu.sync_copy(acc.at[pl.ds(sid * SLAB, SLAB)], buf)
    pltpu.sync_copy(buf, out_hbm.at[cid, pl.ds(sid * SLAB, SLAB)])


def _prep_body(x_ref, w_ref, degc_ref, y_ref, dinv_ref):
    deg = degc_ref[0] + degc_ref[1] + 1.0
    dinv = lax.rsqrt(deg)
    xw = jnp.dot(x_ref[...], w_ref[...], preferred_element_type=jnp.float32)
    y_ref[...] = xw * dinv
    dinv_ref[...] = dinv


def _mid_body(z_ref, y1_ref, dinv_ref, b1_ref, w2_ref, y2_ref):
    dinv = dinv_ref[...]
    h1 = jnp.maximum(
        (z_ref[0] + z_ref[1] + y1_ref[...]) * dinv + b1_ref[...], 0.0)
    y2_ref[...] = jnp.dot(h1, w2_ref[...],
                          preferred_element_type=jnp.float32) * dinv


def _fin_body(z_ref, y2_ref, dinv_ref, b2_ref, batch_ref, wl_ref, bl_ref,
              o_ref):
    h2 = jnp.maximum(
        (z_ref[0] + z_ref[1] + y2_ref[...]) * dinv_ref[...] + b2_ref[...],
        0.0)
    gids = lax.broadcasted_iota(jnp.int32, (NG, NP), 0)
    mask = jnp.where(gids == batch_ref[...], 1.0, 0.0)
    pooled = jnp.dot(mask, h2, preferred_element_type=jnp.float32)
    logits = jnp.dot(pooled, wl_ref[...],
                     preferred_element_type=jnp.float32) + bl_ref[...]
    m = jnp.max(logits, axis=1, keepdims=True)
    s = logits - m
    o_ref[...] = s - jnp.log(jnp.sum(jnp.exp(s), axis=1, keepdims=True))


def _tc(body, out_shape):
    return pl.pallas_call(body, out_shape=out_shape)


def kernel(x, edge_index, batch, W1, b1, W2, b2, Wl, bl):
    f32 = jnp.float32
    src = edge_index[0]
    dst = edge_index[1]
    pad = jnp.full((E_PAD - E,), N, jnp.int32)
    srcs = jnp.concatenate([src, pad]).reshape(NW, EPC, CHUNK)
    dsts = jnp.concatenate([dst, pad]).reshape(NW, EPC, CHUNK)
    x_p = jnp.pad(x, ((0, NP - N), (0, 0)))
    batch_p = jnp.concatenate(
        [batch, jnp.full((NP - N,), NG, jnp.int32)]).reshape(1, NP)
    b1r = b1.reshape(1, H)
    b2r = b2.reshape(1, H)
    blr = bl.reshape(1, C)

    degp = _deg_call(dsts)
    degc = degp.reshape(NC, NP, 1)
    y1, dinv = _tc(_prep_body,
                   (jax.ShapeDtypeStruct((NP, H), f32),
                    jax.ShapeDtypeStruct((NP, 1), f32)))(x_p, W1, degc)
    z1 = _spmm_call(y1, srcs, dsts)
    y2 = _tc(_mid_body, jax.ShapeDtypeStruct((NP, H), f32))(
        z1, y1, dinv, b1r, W2)
    z2 = _spmm_call(y2, srcs, dsts)
    out = _tc(_fin_body, jax.ShapeDtypeStruct((NG, C), f32))(
        z2, y2, dinv, b2r, batch_p, Wl, blr)
    return out
